# Initial kernel scaffold; baseline (speedup 1.0000x reference)
import functools

import jax
import jax.numpy as jnp
from jax import lax
from jax.experimental import pallas as pl
from jax.experimental.pallas import tpu as pltpu
from jax.experimental.pallas import tpu_sc as plsc

_NUM_FIELDS = 13
_VOCAB = 100000
_EMB = 64
_BATCH = 16384
_CAT = _NUM_FIELDS * _EMB
_H0, _H1, _OUT = 512, 256, 128
_TOTAL_ROWS = _BATCH * _NUM_FIELDS

_NC, _NS = 2, 16
_NW = _NC * _NS
_ROWS_PER_W = _TOTAL_ROWS // _NW
_CHUNK = 128
_NCHUNK = _ROWS_PER_W // _CHUNK

_sc_mesh = plsc.VectorSubcoreMesh(core_axis_name="c", subcore_axis_name="s")


@functools.partial(
    pl.kernel,
    out_type=jax.ShapeDtypeStruct((_TOTAL_ROWS, _EMB), jnp.float32),
    mesh=_sc_mesh,
    scratch_types=[
        pltpu.VMEM((_NCHUNK, _CHUNK), jnp.int32),
        pltpu.VMEM((2, _CHUNK, _EMB), jnp.float32),
        pltpu.SemaphoreType.DMA,
        pltpu.SemaphoreType.DMA,
    ],
)
def _sc_gather(table_hbm, idx_hbm, out_hbm, idx_v, rows_v, gsem, ssem):
    wid = lax.axis_index("s") * _NC + lax.axis_index("c")
    pltpu.sync_copy(idx_hbm.at[pl.ds(wid * _NCHUNK, _NCHUNK)], idx_v)

    pltpu.async_copy(table_hbm.at[idx_v.at[0]], rows_v.at[0], gsem)

    def body(j, carry):
        slot = lax.rem(j, 2)
        nslot = 1 - slot

        @pl.when(j + 1 < _NCHUNK)
        def _():
            pltpu.async_copy(table_hbm.at[idx_v.at[j + 1]], rows_v.at[nslot], gsem)

        pltpu.make_async_copy(table_hbm.at[idx_v.at[j]], rows_v.at[slot], gsem).wait()
        pltpu.async_copy(
            rows_v.at[slot],
            out_hbm.at[pl.ds((wid * _NCHUNK + j) * _CHUNK, _CHUNK)],
            ssem,
        ).wait()
        return carry

    lax.fori_loop(0, _NCHUNK, body, 0)


_BM = 1024


def _mlp_body(x_ref, w1_ref, b1_ref, w2_ref, b2_ref, w3_ref, b3_ref, o_ref):
    h = jnp.dot(x_ref[...], w1_ref[...], preferred_element_type=jnp.float32)
    h = jnp.maximum(h + b1_ref[...], 0.0)
    h = jnp.dot(h, w2_ref[...], preferred_element_type=jnp.float32)
    h = jnp.maximum(h + b2_ref[...], 0.0)
    o_ref[...] = jnp.dot(h, w3_ref[...], preferred_element_type=jnp.float32) + b3_ref[...]


def _tc_mlp(x, W1, b1, W2, b2, W3, b3):
    nb = _BATCH // _BM
    return pl.pallas_call(
        _mlp_body,
        grid=(nb,),
        in_specs=[
            pl.BlockSpec((_BM, _CAT), lambda i: (i, 0)),
            pl.BlockSpec((_CAT, _H0), lambda i: (0, 0)),
            pl.BlockSpec((1, _H0), lambda i: (0, 0)),
            pl.BlockSpec((_H0, _H1), lambda i: (0, 0)),
            pl.BlockSpec((1, _H1), lambda i: (0, 0)),
            pl.BlockSpec((_H1, _OUT), lambda i: (0, 0)),
            pl.BlockSpec((1, _OUT), lambda i: (0, 0)),
        ],
        out_specs=pl.BlockSpec((_BM, _OUT), lambda i: (i, 0)),
        out_shape=jax.ShapeDtypeStruct((_BATCH, _OUT), jnp.float32),
    )(x, W1, b1.reshape(1, _H0), W2, b2.reshape(1, _H1), W3, b3.reshape(1, _OUT))


def kernel(user_input, item_input, user_emb, item_emb, W_user, b_user,
           W_item, b_item, W2, b2, W3, b3):
    off = jnp.arange(_NUM_FIELDS, dtype=jnp.int32) * _VOCAB
    u_idx = (user_input.astype(jnp.int32) + off[None, :]).reshape(
        _TOTAL_ROWS // _CHUNK, _CHUNK)
    i_idx = (item_input.astype(jnp.int32) + off[None, :]).reshape(
        _TOTAL_ROWS // _CHUNK, _CHUNK)

    u_cat = _sc_gather(user_emb.reshape(_NUM_FIELDS * _VOCAB, _EMB), u_idx)
    out1 = _tc_mlp(u_cat.reshape(_BATCH, _CAT), W_user, b_user, W2, b2, W3, b3)
    v_cat = _sc_gather(item_emb.reshape(_NUM_FIELDS * _VOCAB, _EMB), i_idx)
    out2 = _tc_mlp(v_cat.reshape(_BATCH, _CAT), W_item, b_item, W2, b2, W3, b3)
    return (out1, out2)


# speedup vs baseline: 1.8069x; 1.8069x over previous
import functools

import jax
import jax.numpy as jnp
from jax import lax
from jax.experimental import pallas as pl
from jax.experimental.pallas import tpu as pltpu
from jax.experimental.pallas import tpu_sc as plsc

_NUM_FIELDS = 13
_VOCAB = 100000
_EMB = 64
_BATCH = 16384
_CAT = _NUM_FIELDS * _EMB
_H0, _H1, _OUT = 512, 256, 128
_TOTAL_ROWS = _BATCH * _NUM_FIELDS

_NC, _NS = 2, 16
_NW = _NC * _NS
_ROWS_PER_W = _TOTAL_ROWS // _NW
_CHUNK = 128
_NCHUNK = _ROWS_PER_W // _CHUNK

_sc_mesh = plsc.VectorSubcoreMesh(core_axis_name="c", subcore_axis_name="s")


@functools.partial(
    pl.kernel,
    out_type=jax.ShapeDtypeStruct((_TOTAL_ROWS, _EMB), jnp.float32),
    mesh=_sc_mesh,
    scratch_types=[
        pltpu.VMEM((_NCHUNK, _CHUNK), jnp.int32),
        pltpu.VMEM((2, _CHUNK, _EMB), jnp.float32),
        pltpu.SemaphoreType.DMA,
        pltpu.SemaphoreType.DMA,
    ],
    compiler_params=pltpu.CompilerParams(use_tc_tiling_on_sc=False),
)
def _sc_gather(table_hbm, idx_hbm, out_hbm, idx_v, rows_v, gsem, ssem):
    wid = lax.axis_index("s") * _NC + lax.axis_index("c")
    pltpu.sync_copy(idx_hbm.at[wid], idx_v)

    pltpu.async_copy(table_hbm.at[idx_v.at[0]], rows_v.at[0], gsem)

    def body(j, carry):
        slot = lax.rem(j, 2)
        nslot = 1 - slot

        @pl.when(j + 1 < _NCHUNK)
        def _():
            pltpu.async_copy(table_hbm.at[idx_v.at[j + 1]], rows_v.at[nslot], gsem)

        pltpu.make_async_copy(table_hbm.at[idx_v.at[j]], rows_v.at[slot], gsem).wait()
        pltpu.async_copy(
            rows_v.at[slot],
            out_hbm.at[pl.ds((wid * _NCHUNK + j) * _CHUNK, _CHUNK)],
            ssem,
        ).wait()
        return carry

    lax.fori_loop(0, _NCHUNK, body, 0)


_BM = 1024


def _mlp_body(x_ref, w1_ref, b1_ref, w2_ref, b2_ref, w3_ref, b3_ref, o_ref):
    h = jnp.dot(x_ref[...], w1_ref[...], preferred_element_type=jnp.float32)
    h = jnp.maximum(h + b1_ref[...], 0.0)
    h = jnp.dot(h, w2_ref[...], preferred_element_type=jnp.float32)
    h = jnp.maximum(h + b2_ref[...], 0.0)
    o_ref[...] = jnp.dot(h, w3_ref[...], preferred_element_type=jnp.float32) + b3_ref[...]


def _tc_mlp(x, W1, b1, W2, b2, W3, b3):
    nb = _BATCH // _BM
    return pl.pallas_call(
        _mlp_body,
        grid=(nb,),
        in_specs=[
            pl.BlockSpec((_BM, _CAT), lambda i: (i, 0)),
            pl.BlockSpec((_CAT, _H0), lambda i: (0, 0)),
            pl.BlockSpec((1, _H0), lambda i: (0, 0)),
            pl.BlockSpec((_H0, _H1), lambda i: (0, 0)),
            pl.BlockSpec((1, _H1), lambda i: (0, 0)),
            pl.BlockSpec((_H1, _OUT), lambda i: (0, 0)),
            pl.BlockSpec((1, _OUT), lambda i: (0, 0)),
        ],
        out_specs=pl.BlockSpec((_BM, _OUT), lambda i: (i, 0)),
        out_shape=jax.ShapeDtypeStruct((_BATCH, _OUT), jnp.float32),
    )(x, W1, b1.reshape(1, _H0), W2, b2.reshape(1, _H1), W3, b3.reshape(1, _OUT))


def kernel(user_input, item_input, user_emb, item_emb, W_user, b_user,
           W_item, b_item, W2, b2, W3, b3):
    off = jnp.arange(_NUM_FIELDS, dtype=jnp.int32) * _VOCAB
    u_idx = (user_input.astype(jnp.int32) + off[None, :]).reshape(
        _NW, _NCHUNK, _CHUNK)
    i_idx = (item_input.astype(jnp.int32) + off[None, :]).reshape(
        _NW, _NCHUNK, _CHUNK)

    u_cat = _sc_gather(user_emb.reshape(_NUM_FIELDS * _VOCAB, _EMB), u_idx)
    out1 = _tc_mlp(u_cat.reshape(_BATCH, _CAT), W_user, b_user, W2, b2, W3, b3)
    v_cat = _sc_gather(item_emb.reshape(_NUM_FIELDS * _VOCAB, _EMB), i_idx)
    out2 = _tc_mlp(v_cat.reshape(_BATCH, _CAT), W_item, b_item, W2, b2, W3, b3)
    return (out1, out2)
